# baseline (speedup 1.0000x reference)
import jax
import jax.numpy as jnp
from jax import lax
from jax.experimental import pallas as pl
from jax.experimental.pallas import tpu as pltpu
from jax.experimental.pallas import tpu_sc as plsc

_LANES = 16
_SPLIT = 104
_NW = 32
_NBUF = 3


def _emb_body(idx_hbm, tok_hbm, pos_hbm, out_hbm, pos_v, idx_v, *bufs):
    rows = bufs[:_NBUF]
    gsA = bufs[_NBUF:2 * _NBUF]
    gsB = bufs[2 * _NBUF:3 * _NBUF]
    osA = bufs[3 * _NBUF:4 * _NBUF]
    osB = bufs[4 * _NBUF:5 * _NBUF]
    isem, psem = bufs[5 * _NBUF:]

    seq, d = pos_v.shape
    nseq = out_hbm.shape[0] // seq // _NW
    wid = lax.axis_index("c") * 16 + lax.axis_index("s")
    seq0 = wid * nseq
    nA, nB = _SPLIT, seq - _SPLIT

    idx_cp = pltpu.async_copy(
        idx_hbm.at[pl.ds(seq0 * seq, nseq * seq)], idx_v, isem)
    pos_cp = pltpu.async_copy(pos_hbm, pos_v, psem)
    idx_cp.wait()

    def fire_gA(c, b):
        pltpu.async_copy(tok_hbm.at[idx_v.at[pl.ds(c * seq, nA)]],
                         rows[b].at[pl.ds(0, nA)], gsA[b])

    def fire_gB(c, b):
        pltpu.async_copy(tok_hbm.at[idx_v.at[pl.ds(c * seq + nA, nB)]],
                         rows[b].at[pl.ds(nA, nB)], gsB[b])

    def wait_gA(b):
        pltpu.make_async_copy(tok_hbm.at[pl.ds(0, nA)],
                              rows[b].at[pl.ds(0, nA)], gsA[b]).wait()

    def wait_gB(b):
        pltpu.make_async_copy(tok_hbm.at[pl.ds(0, nB)],
                              rows[b].at[pl.ds(nA, nB)], gsB[b]).wait()

    def fire_outA(c, b):
        pltpu.async_copy(rows[b].at[pl.ds(0, nA)],
                         out_hbm.at[pl.ds((seq0 + c) * seq, nA)], osA[b])

    def fire_outB(c, b):
        pltpu.async_copy(rows[b].at[pl.ds(nA, nB)],
                         out_hbm.at[pl.ds((seq0 + c) * seq + nA, nB)], osB[b])

    def wait_outA(b):
        pltpu.make_async_copy(rows[b].at[pl.ds(0, nA)],
                              out_hbm.at[pl.ds(0, nA)], osA[b]).wait()

    def wait_outB(b):
        pltpu.make_async_copy(rows[b].at[pl.ds(nA, nB)],
                              out_hbm.at[pl.ds(0, nB)], osB[b]).wait()

    def add_rows(b, lo, hi):
        buf = rows[b]

        def add_row(r, acc):
            for j in range(d // _LANES):
                sl = pl.ds(j * _LANES, _LANES)
                buf[r, sl] = buf[r, sl] + pos_v[r, sl]
            return acc

        lax.fori_loop(lo, hi, add_row, 0)

    fire_gA(0, 0)
    fire_gB(0, 0)
    fire_gA(1, 1)
    fire_gB(1, 1)

    wait_gA(0)
    pos_cp.wait()
    add_rows(0, 0, nA)
    fire_outA(0, 0)
    fire_gA(2, 2)
    wait_gB(0)
    add_rows(0, nA, seq)
    fire_outB(0, 0)
    fire_gB(2, 2)

    wait_gA(1)
    add_rows(1, 0, nA)
    fire_outA(1, 1)
    wait_outA(0)
    fire_gA(3, 0)
    wait_gB(1)
    add_rows(1, nA, seq)
    fire_outB(1, 1)
    wait_outB(0)
    fire_gB(3, 0)

    wait_gA(2)
    add_rows(2, 0, nA)
    fire_outA(2, 2)
    wait_outA(1)
    fire_gA(4, 1)
    wait_gB(2)
    add_rows(2, nA, seq)
    fire_outB(2, 2)
    wait_outB(1)
    fire_gB(4, 1)

    def triple_body(gg, carry):
        c = 3 * gg
        for b in range(_NBUF):
            bn = (b + 2) % _NBUF
            wait_gA(b)
            add_rows(b, 0, nA)
            fire_outA(c + b, b)
            wait_outA(bn)
            fire_gA(c + b + 2, bn)
            wait_gB(b)
            add_rows(b, nA, seq)
            fire_outB(c + b, b)
            wait_outB(bn)
            fire_gB(c + b + 2, bn)
        return carry

    lax.fori_loop(1, (nseq - 2) // 3, triple_body, 0)

    for c, b in ((nseq - 2, 0), (nseq - 1, 1)):
        wait_gA(b)
        add_rows(b, 0, nA)
        fire_outA(c, b)
        wait_gB(b)
        add_rows(b, nA, seq)
        fire_outB(c, b)

    for b in (2, 0, 1):
        wait_outA(b)
        wait_outB(b)


def kernel(inputs, token_table, pos_table):
    b, s = inputs.shape
    v, d = token_table.shape
    nseq = b // _NW
    idx = inputs.astype(jnp.int32).reshape(b * s)
    mesh = plsc.VectorSubcoreMesh(core_axis_name="c", subcore_axis_name="s")
    scratch = [
        pltpu.VMEM((s, d), jnp.float32),
        pltpu.VMEM((nseq * s,), jnp.int32),
    ]
    scratch += [pltpu.VMEM((s, d), jnp.float32) for _ in range(_NBUF)]
    scratch += [pltpu.SemaphoreType.DMA for _ in range(4 * _NBUF + 2)]
    run = pl.kernel(
        _emb_body,
        mesh=mesh,
        out_type=jax.ShapeDtypeStruct((b * s, d), jnp.float32),
        scratch_types=scratch,
    )
    out = run(idx, token_table, pos_table)
    return out.reshape(b, s, d)


# speedup vs baseline: 1.0039x; 1.0039x over previous
import jax
import jax.numpy as jnp
from jax import lax
from jax.experimental import pallas as pl
from jax.experimental.pallas import tpu as pltpu
from jax.experimental.pallas import tpu_sc as plsc

_LANES = 16
_IDXCOLS = 100
_NW = 32
_NBUF = 3


def _emb_body(idx_hbm, tok_hbm, pos_hbm, out_hbm,
              pos_v, idx_v, rows0, rows1, rows2,
              gs0, gs1, gs2, os0, os1, os2, isem, psem):
    seq, d = pos_v.shape
    nseq = out_hbm.shape[0] // seq // _NW
    wid = lax.axis_index("c") * 16 + lax.axis_index("s")
    seq0 = wid * nseq

    idx_cp = pltpu.async_copy(
        idx_hbm.at[pl.ds(seq0 * 2, nseq * 2)], idx_v, isem)
    pos_cp = pltpu.async_copy(pos_hbm, pos_v, psem)
    idx_cp.wait()

    rows = (rows0, rows1, rows2)
    gs = (gs0, gs1, gs2)
    os = (os0, os1, os2)

    def fire_gather(c, b):
        pltpu.async_copy(
            tok_hbm.at[idx_v.at[2 * c]], rows[b].at[pl.ds(0, _IDXCOLS)], gs[b])
        pltpu.async_copy(
            tok_hbm.at[idx_v.at[2 * c + 1]],
            rows[b].at[pl.ds(_IDXCOLS, _IDXCOLS)], gs[b])

    def wait_gather(b):
        pltpu.make_async_copy(tok_hbm.at[pl.ds(0, seq)], rows[b], gs[b]).wait()

    split = 104

    def fire_out_piece(c, b, lo, n):
        pltpu.async_copy(rows[b].at[pl.ds(lo, n)],
                         out_hbm.at[pl.ds((seq0 + c) * seq + lo, n)], os[b])

    def wait_out(b):
        pltpu.make_async_copy(rows[b], out_hbm.at[pl.ds(0, seq)], os[b]).wait()

    def add_rows(b, lo, hi):
        buf = rows[b]

        def add_row(r, acc):
            for j in range(d // _LANES):
                sl = pl.ds(j * _LANES, _LANES)
                buf[r, sl] = buf[r, sl] + pos_v[r, sl]
            return acc

        lax.fori_loop(lo, hi, add_row, 0)

    fire_gather(0, 0)
    fire_gather(1, 1)

    wait_gather(0)
    fire_gather(2, 2)
    pos_cp.wait()
    add_rows(0, 0, split)
    fire_out_piece(0, 0, 0, split)
    add_rows(0, split, seq)
    fire_out_piece(0, 0, split, seq - split)

    wait_gather(1)
    add_rows(1, 0, split)
    fire_out_piece(1, 1, 0, split)
    wait_out(0)
    fire_gather(3, 0)
    add_rows(1, split, seq)
    fire_out_piece(1, 1, split, seq - split)

    wait_gather(2)
    add_rows(2, 0, split)
    fire_out_piece(2, 2, 0, split)
    wait_out(1)
    fire_gather(4, 1)
    add_rows(2, split, seq)
    fire_out_piece(2, 2, split, seq - split)

    def triple_body(gg, carry):
        c = 3 * gg
        for b in range(_NBUF):
            bn = (b + 2) % _NBUF
            wait_gather(b)
            add_rows(b, 0, split)
            fire_out_piece(c + b, b, 0, split)
            wait_out(bn)
            fire_gather(c + b + 2, bn)
            add_rows(b, split, seq)
            fire_out_piece(c + b, b, split, seq - split)
        return carry

    lax.fori_loop(1, (nseq - 2) // 3, triple_body, 0)

    wait_gather(0)
    add_rows(0, 0, split)
    fire_out_piece(nseq - 2, 0, 0, split)
    add_rows(0, split, seq)
    fire_out_piece(nseq - 2, 0, split, seq - split)

    wait_gather(1)
    add_rows(1, 0, split)
    fire_out_piece(nseq - 1, 1, 0, split)
    add_rows(1, split, seq)
    fire_out_piece(nseq - 1, 1, split, seq - split)

    wait_out(2)
    wait_out(0)
    wait_out(1)


def kernel(inputs, token_table, pos_table):
    b, s = inputs.shape
    v, d = token_table.shape
    nseq = b // _NW
    idx = inputs.astype(jnp.int32).reshape(b * s // _IDXCOLS, _IDXCOLS)
    mesh = plsc.VectorSubcoreMesh(core_axis_name="c", subcore_axis_name="s")
    run = pl.kernel(
        _emb_body,
        mesh=mesh,
        out_type=jax.ShapeDtypeStruct((b * s, d), jnp.float32),
        scratch_types=[
            pltpu.VMEM((s, d), jnp.float32),
            pltpu.VMEM((nseq * 2, _IDXCOLS), jnp.int32),
            pltpu.VMEM((s, d), jnp.float32),
            pltpu.VMEM((s, d), jnp.float32),
            pltpu.VMEM((s, d), jnp.float32),
            pltpu.SemaphoreType.DMA,
            pltpu.SemaphoreType.DMA,
            pltpu.SemaphoreType.DMA,
            pltpu.SemaphoreType.DMA,
            pltpu.SemaphoreType.DMA,
            pltpu.SemaphoreType.DMA,
            pltpu.SemaphoreType.DMA,
            pltpu.SemaphoreType.DMA,
        ],
    )
    out = run(idx, token_table, pos_table)
    return out.reshape(b, s, d)
